# baseline (speedup 1.0000x reference)
import functools
import math

import jax
import jax.numpy as jnp
from jax import lax
from jax.experimental import pallas as pl
from jax.experimental.pallas import tpu as pltpu
from jax.experimental.pallas import tpu_sc as plsc

D_MODEL = 64
LANES = 16
SUB = 128
CHUNK_SUBS = 5
CHUNK = SUB * CHUNK_SUBS


def _make_gather(num_idx: int, scale: float):
    info = plsc.get_sparse_core_info()
    nc, ns = info.num_cores, info.num_subcores
    nw = nc * ns
    bpw = num_idx // nw
    assert num_idx % (nw * CHUNK) == 0
    nsub = bpw // SUB
    nchunks = bpw // CHUNK

    mesh = plsc.VectorSubcoreMesh(core_axis_name="c", subcore_axis_name="s")

    @functools.partial(
        pl.kernel,
        mesh=mesh,
        out_type=jax.ShapeDtypeStruct((num_idx, D_MODEL), jnp.float32),
        scratch_types=[
            pltpu.VMEM((nsub, SUB), jnp.int32),
            pltpu.VMEM((CHUNK, D_MODEL), jnp.float32),
            pltpu.SemaphoreType.DMA,
        ],
        compiler_params=pltpu.CompilerParams(use_tc_tiling_on_sc=False),
    )
    def gather_kernel(table_hbm, idx_hbm, out_hbm, idx_v, rows_v, sem):
        wid = lax.axis_index("s") * nc + lax.axis_index("c")
        pltpu.sync_copy(idx_hbm.at[wid], idx_v)

        def chunk_body(g, _):
            copies = []
            for j in range(CHUNK_SUBS):
                copies.append(
                    pltpu.async_copy(
                        table_hbm.at[idx_v.at[g * CHUNK_SUBS + j]],
                        rows_v.at[pl.ds(j * SUB, SUB)],
                        sem,
                    )
                )
            for c in copies:
                c.wait()

            def scale_body(r, _):
                for t in range(D_MODEL // LANES):
                    sl = pl.ds(t * LANES, LANES)
                    rows_v[r, sl] = rows_v[r, sl] * scale
                return _

            lax.fori_loop(0, CHUNK, scale_body, None)
            pltpu.sync_copy(
                rows_v, out_hbm.at[pl.ds(wid * bpw + g * CHUNK, CHUNK)]
            )
            return _

        lax.fori_loop(0, nchunks, chunk_body, None)

    return gather_kernel


def kernel(src, table):
    b, s = src.shape
    num_idx = b * s
    info = plsc.get_sparse_core_info()
    nw = info.num_cores * info.num_subcores
    idx = src.reshape(nw, (num_idx // nw) // SUB, SUB).astype(jnp.int32)
    scale = math.sqrt(table.shape[1])
    out = _make_gather(num_idx, scale)(table, idx)
    return out.reshape(b, s, table.shape[1])


# speedup vs baseline: 1.1068x; 1.1068x over previous
import functools
import math

import jax
import jax.numpy as jnp
from jax import lax
from jax.experimental import pallas as pl
from jax.experimental.pallas import tpu as pltpu
from jax.experimental.pallas import tpu_sc as plsc

D_MODEL = 64
LANES = 16
SUB = 128
CHUNK_SUBS = 5
CHUNK = SUB * CHUNK_SUBS


def _make_gather(num_idx: int, scale: float):
    info = plsc.get_sparse_core_info()
    nc, ns = info.num_cores, info.num_subcores
    nw = nc * ns
    bpw = num_idx // nw
    assert num_idx % (nw * 2 * CHUNK) == 0
    nsub = bpw // SUB
    nchunks = bpw // CHUNK

    mesh = plsc.VectorSubcoreMesh(core_axis_name="c", subcore_axis_name="s")

    @functools.partial(
        pl.kernel,
        mesh=mesh,
        out_type=jax.ShapeDtypeStruct((num_idx, D_MODEL), jnp.float32),
        scratch_types=[
            pltpu.VMEM((nsub, SUB), jnp.int32),
            pltpu.VMEM((CHUNK, D_MODEL), jnp.float32),
            pltpu.VMEM((CHUNK, D_MODEL), jnp.float32),
            pltpu.SemaphoreType.DMA,
            pltpu.SemaphoreType.DMA,
            pltpu.SemaphoreType.DMA,
            pltpu.SemaphoreType.DMA,
        ],
        compiler_params=pltpu.CompilerParams(use_tc_tiling_on_sc=False),
    )
    def gather_kernel(
        table_hbm, idx_hbm, out_hbm, idx_v, rows0, rows1, gsem0, gsem1, wsem0, wsem1
    ):
        wid = lax.axis_index("s") * nc + lax.axis_index("c")
        pltpu.sync_copy(idx_hbm.at[wid], idx_v)

        def fire_gathers(g, rows, gsem):
            for j in range(CHUNK_SUBS):
                pltpu.async_copy(
                    table_hbm.at[idx_v.at[g * CHUNK_SUBS + j]],
                    rows.at[pl.ds(j * SUB, SUB)],
                    gsem,
                )

        def wait_gathers(rows, gsem):
            for j in range(CHUNK_SUBS):
                pltpu.make_async_copy(
                    table_hbm.at[idx_v.at[j]],
                    rows.at[pl.ds(j * SUB, SUB)],
                    gsem,
                ).wait()

        def scale_rows(rows):
            @plsc.parallel_loop(0, CHUNK, unroll=4)
            def _(r):
                for t in range(D_MODEL // LANES):
                    sl = pl.ds(t * LANES, LANES)
                    rows[r, sl] = rows[r, sl] * scale

        def out_slice(g):
            return out_hbm.at[pl.ds(wid * bpw + g * CHUNK, CHUNK)]

        def fire_write(g, rows, wsem):
            pltpu.async_copy(rows, out_slice(g), wsem)

        def wait_write(rows, wsem):
            pltpu.make_async_copy(rows, out_slice(0), wsem).wait()

        fire_gathers(0, rows0, gsem0)

        @pl.loop(0, nchunks, step=2)
        def _(a):
            @pl.when(a > 0)
            def _():
                wait_write(rows1, wsem1)

            fire_gathers(a + 1, rows1, gsem1)
            wait_gathers(rows0, gsem0)
            scale_rows(rows0)
            fire_write(a, rows0, wsem0)
            wait_write(rows0, wsem0)

            @pl.when(a + 2 < nchunks)
            def _():
                fire_gathers(a + 2, rows0, gsem0)

            wait_gathers(rows1, gsem1)
            scale_rows(rows1)
            fire_write(a + 1, rows1, wsem1)

        wait_write(rows1, wsem1)

    return gather_kernel


def kernel(src, table):
    b, s = src.shape
    num_idx = b * s
    info = plsc.get_sparse_core_info()
    nw = info.num_cores * info.num_subcores
    idx = src.reshape(nw, (num_idx // nw) // SUB, SUB).astype(jnp.int32)
    scale = math.sqrt(table.shape[1])
    out = _make_gather(num_idx, scale)(table, idx)
    return out.reshape(b, s, table.shape[1])
